# baseline (speedup 1.0000x reference)
import functools

import jax
import jax.numpy as jnp
from jax import lax
from jax.experimental import pallas as pl
from jax.experimental.pallas import tpu as pltpu
from jax.experimental.pallas import tpu_sc as plsc

NUM_NODES = 10000
MEMORY_DIM = 64
MSG_DIM = 64
NODE_DIM = 256
EDGE_DIM = 16
TIME_DIM = 16
STATIC_DIM = 64
DYNAMIC_DIM = 64
EMB_DIM = 32
B = 2048
E = 2 * B

NC, NS = 2, 16
NW = NC * NS
BPW = B // NW

TB_A = 128
TB_B = 256


def _sc_gather_body(mem_hbm, emb_hbm, src_hbm, dst_hbm,
                    out_sm, out_dm, out_se, out_de,
                    idx_s, idx_d, rows_m, rows_m2, rows_e, rows_e2,
                    sem, sem2):
    wid = lax.axis_index("s") * NC + lax.axis_index("c")
    base = wid * BPW
    pltpu.sync_copy(src_hbm.at[pl.ds(base, BPW)], idx_s)
    pltpu.sync_copy(dst_hbm.at[pl.ds(base, BPW)], idx_d)
    c1 = pltpu.async_copy(mem_hbm.at[idx_s], rows_m, sem)
    c2 = pltpu.async_copy(mem_hbm.at[idx_d], rows_m2, sem)
    c3 = pltpu.async_copy(emb_hbm.at[idx_s], rows_e, sem)
    c4 = pltpu.async_copy(emb_hbm.at[idx_d], rows_e2, sem)
    c1.wait()
    o1 = pltpu.async_copy(rows_m, out_sm.at[pl.ds(base, BPW)], sem2)
    c2.wait()
    o2 = pltpu.async_copy(rows_m2, out_dm.at[pl.ds(base, BPW)], sem2)
    c3.wait()
    o3 = pltpu.async_copy(rows_e, out_se.at[pl.ds(base, BPW)], sem2)
    c4.wait()
    o4 = pltpu.async_copy(rows_e2, out_de.at[pl.ds(base, BPW)], sem2)
    o1.wait()
    o2.wait()
    o3.wait()
    o4.wait()


def _sc_gather(memory, emb_table, src, dst):
    mesh = plsc.VectorSubcoreMesh(core_axis_name="c", subcore_axis_name="s")
    f32 = jnp.float32
    call = pl.kernel(
        _sc_gather_body,
        mesh=mesh,
        compiler_params=pltpu.CompilerParams(use_tc_tiling_on_sc=False),
        out_type=(
            jax.ShapeDtypeStruct((B, MEMORY_DIM), f32),
            jax.ShapeDtypeStruct((B, MEMORY_DIM), f32),
            jax.ShapeDtypeStruct((B, EMB_DIM), f32),
            jax.ShapeDtypeStruct((B, EMB_DIM), f32),
        ),
        scratch_types=[
            pltpu.VMEM((BPW,), jnp.int32),
            pltpu.VMEM((BPW,), jnp.int32),
            pltpu.VMEM((BPW, MEMORY_DIM), f32),
            pltpu.VMEM((BPW, MEMORY_DIM), f32),
            pltpu.VMEM((BPW, EMB_DIM), f32),
            pltpu.VMEM((BPW, EMB_DIM), f32),
            pltpu.SemaphoreType.DMA,
            pltpu.SemaphoreType.DMA,
        ],
    )
    return call(memory, emb_table, src, dst)


N_A = B // TB_A
N_B = B // TB_B


def _sigmoid(x):
    return 1.0 / (1.0 + jnp.exp(-x))


def _fused_body(t_ref, ea_a, sm_a, dm_a,
                sc_ref, dc_ref, sm_b, dm_b, ss_ref, ds_ref, sy_ref, dy_ref,
                se_ref, de_ref, ea_b, sr_ref, dr_ref, tr_ref,
                wt_ref, bt_ref, w1_ref, b1_ref, tf_ref, b0_ref,
                gwi_ref, gwh_ref, gbi_ref, gbh_ref,
                mw1_ref, mb1_ref, mw2_ref, mb2_ref, pw_ref, pb_ref,
                out_ref, msg_scr):
    i = pl.program_id(0)

    @pl.when(i < N_A)
    def _msg_phase():
        tf = t_ref[...]
        te = jnp.cos(tf * wt_ref[...] + bt_ref[...])
        raw = jnp.concatenate([ea_a[...], te], axis=1)
        h = jnp.maximum(
            jnp.dot(raw, w1_ref[...], preferred_element_type=jnp.float32)
            + b1_ref[...], 0.0)
        sm = sm_a[...]
        dm = dm_a[...]
        x2 = jnp.concatenate(
            [jnp.concatenate([sm, dm], axis=1),
             jnp.concatenate([dm, sm], axis=1)], axis=0)
        h2b = jnp.concatenate([h, h], axis=0).astype(jnp.bfloat16)
        x2b = x2.astype(jnp.bfloat16)
        outer = jnp.concatenate(
            [h2b[:, c:c + 1] * x2b for c in range(64)],
            axis=1)
        acc = (jnp.dot(outer, tf_ref[...], preferred_element_type=jnp.float32)
               + jnp.dot(x2, b0_ref[...], preferred_element_type=jnp.float32))
        base = i * TB_A
        msg_scr[0, pl.ds(base, TB_A), :] = acc[:TB_A].astype(jnp.bfloat16)
        msg_scr[1, pl.ds(base, TB_A), :] = acc[TB_A:].astype(jnp.bfloat16)

    @pl.when(i >= N_A)
    def _tail_phase():
        irow = lax.broadcasted_iota(jnp.int32, (1, B), 1)
        key_s = tr_ref[...] * E + irow
        key_d = key_s + B
        src_row = sr_ref[...]
        dst_row = dr_ref[...]
        msg_s = msg_scr[0]
        msg_d = msg_scr[1]

        def agg_for(idx_col):
            m1 = jnp.where(src_row == idx_col, key_s, -1)
            m2 = jnp.where(dst_row == idx_col, key_d, -1)
            wk = jnp.maximum(jnp.max(m1, axis=1, keepdims=True),
                             jnp.max(m2, axis=1, keepdims=True))
            oh_s = (key_s == wk).astype(jnp.bfloat16)
            oh_d = (key_d == wk).astype(jnp.bfloat16)
            return (jnp.dot(oh_s, msg_s, preferred_element_type=jnp.float32)
                    + jnp.dot(oh_d, msg_d,
                              preferred_element_type=jnp.float32))

        def gru(agg, mem):
            gi = jnp.dot(agg, gwi_ref[...],
                         preferred_element_type=jnp.float32) + gbi_ref[...]
            gh = jnp.dot(mem, gwh_ref[...],
                         preferred_element_type=jnp.float32) + gbh_ref[...]
            r = _sigmoid(gi[:, :64] + gh[:, :64])
            z = _sigmoid(gi[:, 64:128] + gh[:, 64:128])
            n = jnp.tanh(gi[:, 128:] + r * gh[:, 128:])
            return (1.0 - z) * n + z * mem

        def mlp(xfull):
            h1 = jnp.maximum(
                jnp.dot(xfull.astype(jnp.bfloat16), mw1_ref[...],
                        preferred_element_type=jnp.float32) + mb1_ref[...],
                0.0)
            return (jnp.dot(h1.astype(jnp.bfloat16), mw2_ref[...],
                            preferred_element_type=jnp.float32) + mb2_ref[...])

        agg_s = agg_for(sc_ref[...])
        agg_d = agg_for(dc_ref[...])
        mem_s = gru(agg_s, sm_b[...])
        mem_d = gru(agg_d, dm_b[...])
        s_full = jnp.concatenate(
            [mem_s, ss_ref[...], se_ref[...], sy_ref[...]], axis=1)
        d_full = jnp.concatenate(
            [mem_d, ds_ref[...], de_ref[...], dy_ref[...]], axis=1)
        cat = jnp.concatenate([mlp(s_full), mlp(d_full), ea_b[...]], axis=1)
        out_ref[...] = (jnp.dot(cat, pw_ref[...],
                                preferred_element_type=jnp.float32)
                        + pb_ref[...])


def _fused(t2d, edge_attr, src_m, dst_m,
           src_col, dst_col, src_static, dst_static, src_dyn, dst_dyn,
           src_e, dst_e, src_row, dst_row, t_row,
           wt, bt, en_w1, en_b1, tflat, b0,
           gru_wi, gru_wh, gru_bi, gru_bh,
           mlp_w1, mlp_b1, mlp_w2, mlp_b2, pred_w, pred_b):
    grid = (N_A + N_B,)

    def tile_a(d):
        return pl.BlockSpec((TB_A, d),
                            lambda i: (jnp.where(i < N_A, i, 0), 0))

    def tile_b(d):
        return pl.BlockSpec((TB_B, d),
                            lambda i: (jnp.where(i < N_A, 0, i - N_A), 0))

    full = lambda r, c: pl.BlockSpec((r, c), lambda i: (0, 0))
    in_dim = MEMORY_DIM + STATIC_DIM + EMB_DIM + DYNAMIC_DIM
    return pl.pallas_call(
        _fused_body,
        grid=grid,
        in_specs=[
            tile_a(1), tile_a(EDGE_DIM), tile_a(MEMORY_DIM),
            tile_a(MEMORY_DIM),
            tile_b(1), tile_b(1), tile_b(MEMORY_DIM), tile_b(MEMORY_DIM),
            tile_b(STATIC_DIM), tile_b(STATIC_DIM),
            tile_b(DYNAMIC_DIM), tile_b(DYNAMIC_DIM),
            tile_b(EMB_DIM), tile_b(EMB_DIM), tile_b(EDGE_DIM),
            full(1, B), full(1, B), full(1, B),
            full(1, TIME_DIM), full(1, TIME_DIM),
            full(2 * TIME_DIM, 64), full(1, 64),
            full(64 * 128, MSG_DIM), full(2 * MEMORY_DIM, MSG_DIM),
            full(MSG_DIM, 3 * MEMORY_DIM), full(MEMORY_DIM, 3 * MEMORY_DIM),
            full(1, 3 * MEMORY_DIM), full(1, 3 * MEMORY_DIM),
            full(in_dim, NODE_DIM), full(1, NODE_DIM),
            full(NODE_DIM, NODE_DIM), full(1, NODE_DIM),
            full(2 * NODE_DIM + EDGE_DIM, 1), full(1, 1),
        ],
        out_specs=pl.BlockSpec((TB_B, 1),
                               lambda i: (jnp.where(i < N_A, 0, i - N_A), 0)),
        out_shape=jax.ShapeDtypeStruct((B, 1), jnp.float32),
        scratch_shapes=[pltpu.VMEM((2, B, MSG_DIM), jnp.bfloat16)],
    )(t2d, edge_attr, src_m, dst_m,
      src_col, dst_col, src_m, dst_m, src_static, dst_static,
      src_dyn, dst_dyn, src_e, dst_e, edge_attr,
      src_row, dst_row, t_row,
      wt, bt, en_w1, en_b1, tflat, b0,
      gru_wi, gru_wh, gru_bi, gru_bh,
      mlp_w1, mlp_b1, mlp_w2, mlp_b2, pred_w, pred_b)


def kernel(src, dst, t, edge_attr, src_static, dst_static, src_dynamic,
           dst_dynamic, memory, last_update, w_time, b_time, en_w1, en_b1,
           en_w2, en_b2, gru_wi, gru_wh, gru_bi, gru_bh, emb_table,
           mlp_w1, mlp_b1, mlp_w2, mlp_b2, pred_w, pred_b):
    src = src.astype(jnp.int32)
    dst = dst.astype(jnp.int32)
    t = t.astype(jnp.int32)

    tflat = en_w2.astype(jnp.bfloat16).reshape(64 * 128, MSG_DIM)
    b0 = en_b2.reshape(2 * MEMORY_DIM, MSG_DIM)

    src_m, dst_m, src_e, dst_e = _sc_gather(memory, emb_table, src, dst)

    pred = _fused(t.astype(jnp.float32).reshape(B, 1), edge_attr,
                  src_m, dst_m,
                  src.reshape(B, 1), dst.reshape(B, 1),
                  src_static, dst_static, src_dynamic, dst_dynamic,
                  src_e, dst_e,
                  src.reshape(1, B), dst.reshape(1, B), t.reshape(1, B),
                  w_time, b_time.reshape(1, TIME_DIM),
                  en_w1, en_b1.reshape(1, 64), tflat, b0,
                  gru_wi, gru_wh, gru_bi.reshape(1, 3 * MEMORY_DIM),
                  gru_bh.reshape(1, 3 * MEMORY_DIM),
                  mlp_w1.astype(jnp.bfloat16), mlp_b1.reshape(1, NODE_DIM),
                  mlp_w2.astype(jnp.bfloat16), mlp_b2.reshape(1, NODE_DIM),
                  pred_w, pred_b.reshape(1, 1))
    return pred


# speedup vs baseline: 1.0752x; 1.0752x over previous
import functools

import jax
import jax.numpy as jnp
from jax import lax
from jax.experimental import pallas as pl
from jax.experimental.pallas import tpu as pltpu
from jax.experimental.pallas import tpu_sc as plsc

NUM_NODES = 10000
MEMORY_DIM = 64
MSG_DIM = 64
NODE_DIM = 256
EDGE_DIM = 16
TIME_DIM = 16
STATIC_DIM = 64
DYNAMIC_DIM = 64
EMB_DIM = 32
B = 2048
E = 2 * B

NC, NS = 2, 16
NW = NC * NS
BPW = B // NW

TB_A = 256
TB_B = 512


def _sc_gather_body(mem_hbm, emb_hbm, src_hbm, dst_hbm,
                    out_sm, out_dm, out_se, out_de,
                    idx_s, idx_d, rows_m, rows_m2, rows_e, rows_e2,
                    sem, sem2):
    wid = lax.axis_index("s") * NC + lax.axis_index("c")
    base = wid * BPW
    pltpu.sync_copy(src_hbm.at[pl.ds(base, BPW)], idx_s)
    pltpu.sync_copy(dst_hbm.at[pl.ds(base, BPW)], idx_d)
    c1 = pltpu.async_copy(mem_hbm.at[idx_s], rows_m, sem)
    c2 = pltpu.async_copy(mem_hbm.at[idx_d], rows_m2, sem)
    c3 = pltpu.async_copy(emb_hbm.at[idx_s], rows_e, sem)
    c4 = pltpu.async_copy(emb_hbm.at[idx_d], rows_e2, sem)
    c1.wait()
    o1 = pltpu.async_copy(rows_m, out_sm.at[pl.ds(base, BPW)], sem2)
    c2.wait()
    o2 = pltpu.async_copy(rows_m2, out_dm.at[pl.ds(base, BPW)], sem2)
    c3.wait()
    o3 = pltpu.async_copy(rows_e, out_se.at[pl.ds(base, BPW)], sem2)
    c4.wait()
    o4 = pltpu.async_copy(rows_e2, out_de.at[pl.ds(base, BPW)], sem2)
    o1.wait()
    o2.wait()
    o3.wait()
    o4.wait()


def _sc_gather(memory, emb_table, src, dst):
    mesh = plsc.VectorSubcoreMesh(core_axis_name="c", subcore_axis_name="s")
    f32 = jnp.float32
    call = pl.kernel(
        _sc_gather_body,
        mesh=mesh,
        compiler_params=pltpu.CompilerParams(use_tc_tiling_on_sc=False),
        out_type=(
            jax.ShapeDtypeStruct((B, MEMORY_DIM), f32),
            jax.ShapeDtypeStruct((B, MEMORY_DIM), f32),
            jax.ShapeDtypeStruct((B, EMB_DIM), f32),
            jax.ShapeDtypeStruct((B, EMB_DIM), f32),
        ),
        scratch_types=[
            pltpu.VMEM((BPW,), jnp.int32),
            pltpu.VMEM((BPW,), jnp.int32),
            pltpu.VMEM((BPW, MEMORY_DIM), f32),
            pltpu.VMEM((BPW, MEMORY_DIM), f32),
            pltpu.VMEM((BPW, EMB_DIM), f32),
            pltpu.VMEM((BPW, EMB_DIM), f32),
            pltpu.SemaphoreType.DMA,
            pltpu.SemaphoreType.DMA,
        ],
    )
    return call(memory, emb_table, src, dst)


N_A = B // TB_A
N_B = B // TB_B


def _sigmoid(x):
    return 1.0 / (1.0 + jnp.exp(-x))


def _fused_body(t_ref, ea_a, sm_a, dm_a,
                sc_ref, dc_ref, sm_b, dm_b, ss_ref, ds_ref, sy_ref, dy_ref,
                se_ref, de_ref, ea_b, sr_ref, dr_ref, tr_ref,
                wt_ref, bt_ref, w1_ref, b1_ref, tf_ref, b0_ref,
                gwi_ref, gwh_ref, gbi_ref, gbh_ref,
                mw1_ref, mb1_ref, mw2_ref, mb2_ref, pw_ref, pb_ref,
                out_ref, msg_scr):
    i = pl.program_id(0)

    @pl.when(i < N_A)
    def _msg_phase():
        tf = t_ref[...]
        te = jnp.cos(tf * wt_ref[...] + bt_ref[...])
        raw = jnp.concatenate([ea_a[...], te], axis=1)
        h = jnp.maximum(
            jnp.dot(raw, w1_ref[...], preferred_element_type=jnp.float32)
            + b1_ref[...], 0.0)
        sm = sm_a[...]
        dm = dm_a[...]
        x2 = jnp.concatenate(
            [jnp.concatenate([sm, dm], axis=1),
             jnp.concatenate([dm, sm], axis=1)], axis=0)
        h2b = jnp.concatenate([h, h], axis=0).astype(jnp.bfloat16)
        x2b = x2.astype(jnp.bfloat16)
        outer = jnp.concatenate(
            [h2b[:, c:c + 1] * x2b for c in range(64)],
            axis=1)
        acc = (jnp.dot(outer, tf_ref[...], preferred_element_type=jnp.float32)
               + jnp.dot(x2, b0_ref[...], preferred_element_type=jnp.float32))
        base = i * TB_A
        msg_scr[0, pl.ds(base, TB_A), :] = acc[:TB_A].astype(jnp.bfloat16)
        msg_scr[1, pl.ds(base, TB_A), :] = acc[TB_A:].astype(jnp.bfloat16)

    @pl.when(i >= N_A)
    def _tail_phase():
        irow = lax.broadcasted_iota(jnp.int32, (1, B), 1)
        key_s = tr_ref[...] * E + irow
        key_d = key_s + B
        src_row = sr_ref[...]
        dst_row = dr_ref[...]
        msg_s = msg_scr[0]
        msg_d = msg_scr[1]

        def agg_for(idx_col):
            m1 = jnp.where(src_row == idx_col, key_s, -1)
            m2 = jnp.where(dst_row == idx_col, key_d, -1)
            wk = jnp.maximum(jnp.max(m1, axis=1, keepdims=True),
                             jnp.max(m2, axis=1, keepdims=True))
            oh_s = (key_s == wk).astype(jnp.bfloat16)
            oh_d = (key_d == wk).astype(jnp.bfloat16)
            return (jnp.dot(oh_s, msg_s, preferred_element_type=jnp.float32)
                    + jnp.dot(oh_d, msg_d,
                              preferred_element_type=jnp.float32))

        def gru(agg, mem):
            gi = jnp.dot(agg, gwi_ref[...],
                         preferred_element_type=jnp.float32) + gbi_ref[...]
            gh = jnp.dot(mem, gwh_ref[...],
                         preferred_element_type=jnp.float32) + gbh_ref[...]
            r = _sigmoid(gi[:, :64] + gh[:, :64])
            z = _sigmoid(gi[:, 64:128] + gh[:, 64:128])
            n = jnp.tanh(gi[:, 128:] + r * gh[:, 128:])
            return (1.0 - z) * n + z * mem

        def mlp(xfull):
            h1 = jnp.maximum(
                jnp.dot(xfull.astype(jnp.bfloat16), mw1_ref[...],
                        preferred_element_type=jnp.float32) + mb1_ref[...],
                0.0)
            return (jnp.dot(h1.astype(jnp.bfloat16), mw2_ref[...],
                            preferred_element_type=jnp.float32) + mb2_ref[...])

        agg_s = agg_for(sc_ref[...])
        agg_d = agg_for(dc_ref[...])
        mem_s = gru(agg_s, sm_b[...])
        mem_d = gru(agg_d, dm_b[...])
        s_full = jnp.concatenate(
            [mem_s, ss_ref[...], se_ref[...], sy_ref[...]], axis=1)
        d_full = jnp.concatenate(
            [mem_d, ds_ref[...], de_ref[...], dy_ref[...]], axis=1)
        cat = jnp.concatenate([mlp(s_full), mlp(d_full), ea_b[...]], axis=1)
        out_ref[...] = (jnp.dot(cat, pw_ref[...],
                                preferred_element_type=jnp.float32)
                        + pb_ref[...])


def _fused(t2d, edge_attr, src_m, dst_m,
           src_col, dst_col, src_static, dst_static, src_dyn, dst_dyn,
           src_e, dst_e, src_row, dst_row, t_row,
           wt, bt, en_w1, en_b1, tflat, b0,
           gru_wi, gru_wh, gru_bi, gru_bh,
           mlp_w1, mlp_b1, mlp_w2, mlp_b2, pred_w, pred_b):
    grid = (N_A + N_B,)

    def tile_a(d):
        return pl.BlockSpec((TB_A, d),
                            lambda i: (jnp.where(i < N_A, i, 0), 0))

    def tile_b(d):
        return pl.BlockSpec((TB_B, d),
                            lambda i: (jnp.where(i < N_A, 0, i - N_A), 0))

    full = lambda r, c: pl.BlockSpec((r, c), lambda i: (0, 0))
    in_dim = MEMORY_DIM + STATIC_DIM + EMB_DIM + DYNAMIC_DIM
    return pl.pallas_call(
        _fused_body,
        grid=grid,
        in_specs=[
            tile_a(1), tile_a(EDGE_DIM), tile_a(MEMORY_DIM),
            tile_a(MEMORY_DIM),
            tile_b(1), tile_b(1), tile_b(MEMORY_DIM), tile_b(MEMORY_DIM),
            tile_b(STATIC_DIM), tile_b(STATIC_DIM),
            tile_b(DYNAMIC_DIM), tile_b(DYNAMIC_DIM),
            tile_b(EMB_DIM), tile_b(EMB_DIM), tile_b(EDGE_DIM),
            full(1, B), full(1, B), full(1, B),
            full(1, TIME_DIM), full(1, TIME_DIM),
            full(2 * TIME_DIM, 64), full(1, 64),
            full(64 * 128, MSG_DIM), full(2 * MEMORY_DIM, MSG_DIM),
            full(MSG_DIM, 3 * MEMORY_DIM), full(MEMORY_DIM, 3 * MEMORY_DIM),
            full(1, 3 * MEMORY_DIM), full(1, 3 * MEMORY_DIM),
            full(in_dim, NODE_DIM), full(1, NODE_DIM),
            full(NODE_DIM, NODE_DIM), full(1, NODE_DIM),
            full(2 * NODE_DIM + EDGE_DIM, 1), full(1, 1),
        ],
        out_specs=pl.BlockSpec((TB_B, 1),
                               lambda i: (jnp.where(i < N_A, 0, i - N_A), 0)),
        out_shape=jax.ShapeDtypeStruct((B, 1), jnp.float32),
        scratch_shapes=[pltpu.VMEM((2, B, MSG_DIM), jnp.bfloat16)],
    )(t2d, edge_attr, src_m, dst_m,
      src_col, dst_col, src_m, dst_m, src_static, dst_static,
      src_dyn, dst_dyn, src_e, dst_e, edge_attr,
      src_row, dst_row, t_row,
      wt, bt, en_w1, en_b1, tflat, b0,
      gru_wi, gru_wh, gru_bi, gru_bh,
      mlp_w1, mlp_b1, mlp_w2, mlp_b2, pred_w, pred_b)


def kernel(src, dst, t, edge_attr, src_static, dst_static, src_dynamic,
           dst_dynamic, memory, last_update, w_time, b_time, en_w1, en_b1,
           en_w2, en_b2, gru_wi, gru_wh, gru_bi, gru_bh, emb_table,
           mlp_w1, mlp_b1, mlp_w2, mlp_b2, pred_w, pred_b):
    src = src.astype(jnp.int32)
    dst = dst.astype(jnp.int32)
    t = t.astype(jnp.int32)

    tflat = en_w2.astype(jnp.bfloat16).reshape(64 * 128, MSG_DIM)
    b0 = en_b2.reshape(2 * MEMORY_DIM, MSG_DIM)

    src_m, dst_m, src_e, dst_e = _sc_gather(memory, emb_table, src, dst)

    pred = _fused(t.astype(jnp.float32).reshape(B, 1), edge_attr,
                  src_m, dst_m,
                  src.reshape(B, 1), dst.reshape(B, 1),
                  src_static, dst_static, src_dynamic, dst_dynamic,
                  src_e, dst_e,
                  src.reshape(1, B), dst.reshape(1, B), t.reshape(1, B),
                  w_time, b_time.reshape(1, TIME_DIM),
                  en_w1, en_b1.reshape(1, 64), tflat, b0,
                  gru_wi, gru_wh, gru_bi.reshape(1, 3 * MEMORY_DIM),
                  gru_bh.reshape(1, 3 * MEMORY_DIM),
                  mlp_w1.astype(jnp.bfloat16), mlp_b1.reshape(1, NODE_DIM),
                  mlp_w2.astype(jnp.bfloat16), mlp_b2.reshape(1, NODE_DIM),
                  pred_w, pred_b.reshape(1, 1))
    return pred


# speedup vs baseline: 1.0974x; 1.0207x over previous
import functools

import jax
import jax.numpy as jnp
from jax import lax
from jax.experimental import pallas as pl
from jax.experimental.pallas import tpu as pltpu
from jax.experimental.pallas import tpu_sc as plsc

NUM_NODES = 10000
MEMORY_DIM = 64
MSG_DIM = 64
NODE_DIM = 256
EDGE_DIM = 16
TIME_DIM = 16
STATIC_DIM = 64
DYNAMIC_DIM = 64
EMB_DIM = 32
B = 2048
E = 2 * B

NC, NS = 2, 16
NW = NC * NS
BPW = B // NW

TB_A = 512
TB_B = 512


def _sc_gather_body(mem_hbm, emb_hbm, src_hbm, dst_hbm,
                    out_sm, out_dm, out_se, out_de,
                    idx_s, idx_d, rows_m, rows_m2, rows_e, rows_e2,
                    sem, sem2):
    wid = lax.axis_index("s") * NC + lax.axis_index("c")
    base = wid * BPW
    pltpu.sync_copy(src_hbm.at[pl.ds(base, BPW)], idx_s)
    pltpu.sync_copy(dst_hbm.at[pl.ds(base, BPW)], idx_d)
    c1 = pltpu.async_copy(mem_hbm.at[idx_s], rows_m, sem)
    c2 = pltpu.async_copy(mem_hbm.at[idx_d], rows_m2, sem)
    c3 = pltpu.async_copy(emb_hbm.at[idx_s], rows_e, sem)
    c4 = pltpu.async_copy(emb_hbm.at[idx_d], rows_e2, sem)
    c1.wait()
    o1 = pltpu.async_copy(rows_m, out_sm.at[pl.ds(base, BPW)], sem2)
    c2.wait()
    o2 = pltpu.async_copy(rows_m2, out_dm.at[pl.ds(base, BPW)], sem2)
    c3.wait()
    o3 = pltpu.async_copy(rows_e, out_se.at[pl.ds(base, BPW)], sem2)
    c4.wait()
    o4 = pltpu.async_copy(rows_e2, out_de.at[pl.ds(base, BPW)], sem2)
    o1.wait()
    o2.wait()
    o3.wait()
    o4.wait()


def _sc_gather(memory, emb_table, src, dst):
    mesh = plsc.VectorSubcoreMesh(core_axis_name="c", subcore_axis_name="s")
    f32 = jnp.float32
    call = pl.kernel(
        _sc_gather_body,
        mesh=mesh,
        compiler_params=pltpu.CompilerParams(use_tc_tiling_on_sc=False),
        out_type=(
            jax.ShapeDtypeStruct((B, MEMORY_DIM), f32),
            jax.ShapeDtypeStruct((B, MEMORY_DIM), f32),
            jax.ShapeDtypeStruct((B, EMB_DIM), f32),
            jax.ShapeDtypeStruct((B, EMB_DIM), f32),
        ),
        scratch_types=[
            pltpu.VMEM((BPW,), jnp.int32),
            pltpu.VMEM((BPW,), jnp.int32),
            pltpu.VMEM((BPW, MEMORY_DIM), f32),
            pltpu.VMEM((BPW, MEMORY_DIM), f32),
            pltpu.VMEM((BPW, EMB_DIM), f32),
            pltpu.VMEM((BPW, EMB_DIM), f32),
            pltpu.SemaphoreType.DMA,
            pltpu.SemaphoreType.DMA,
        ],
    )
    return call(memory, emb_table, src, dst)


N_A = B // TB_A
N_B = B // TB_B


def _sigmoid(x):
    return 1.0 / (1.0 + jnp.exp(-x))


def _fused_body(t_ref, ea_a, sm_a, dm_a,
                sc_ref, dc_ref, sm_b, dm_b, ss_ref, ds_ref, sy_ref, dy_ref,
                se_ref, de_ref, ea_b, sr_ref, dr_ref, tr_ref,
                wt_ref, bt_ref, w1_ref, b1_ref, tf_ref, b0_ref,
                gwi_ref, gwh_ref, gbi_ref, gbh_ref,
                mw1_ref, mb1_ref, mw2_ref, mb2_ref, pw_ref, pb_ref,
                out_ref, msg_scr):
    i = pl.program_id(0)

    @pl.when(i < N_A)
    def _msg_phase():
        tf = t_ref[...]
        te = jnp.cos(tf * wt_ref[...] + bt_ref[...])
        raw = jnp.concatenate([ea_a[...], te], axis=1)
        h = jnp.maximum(
            jnp.dot(raw, w1_ref[...], preferred_element_type=jnp.float32)
            + b1_ref[...], 0.0)
        sm = sm_a[...]
        dm = dm_a[...]
        x2 = jnp.concatenate(
            [jnp.concatenate([sm, dm], axis=1),
             jnp.concatenate([dm, sm], axis=1)], axis=0)
        h2b = jnp.concatenate([h, h], axis=0).astype(jnp.bfloat16)
        x2b = x2.astype(jnp.bfloat16)
        outer = jnp.concatenate(
            [h2b[:, c:c + 1] * x2b for c in range(64)],
            axis=1)
        acc = (jnp.dot(outer, tf_ref[...], preferred_element_type=jnp.float32)
               + jnp.dot(x2, b0_ref[...], preferred_element_type=jnp.float32))
        base = i * TB_A
        msg_scr[0, pl.ds(base, TB_A), :] = acc[:TB_A].astype(jnp.bfloat16)
        msg_scr[1, pl.ds(base, TB_A), :] = acc[TB_A:].astype(jnp.bfloat16)

    @pl.when(i >= N_A)
    def _tail_phase():
        irow = lax.broadcasted_iota(jnp.int32, (1, B), 1)
        key_s = tr_ref[...] * E + irow
        key_d = key_s + B
        src_row = sr_ref[...]
        dst_row = dr_ref[...]
        msg_s = msg_scr[0]
        msg_d = msg_scr[1]

        def agg_for(idx_col):
            m1 = jnp.where(src_row == idx_col, key_s, -1)
            m2 = jnp.where(dst_row == idx_col, key_d, -1)
            wk = jnp.maximum(jnp.max(m1, axis=1, keepdims=True),
                             jnp.max(m2, axis=1, keepdims=True))
            oh_s = (key_s == wk).astype(jnp.bfloat16)
            oh_d = (key_d == wk).astype(jnp.bfloat16)
            return (jnp.dot(oh_s, msg_s, preferred_element_type=jnp.float32)
                    + jnp.dot(oh_d, msg_d,
                              preferred_element_type=jnp.float32))

        def gru(agg, mem):
            gi = jnp.dot(agg, gwi_ref[...],
                         preferred_element_type=jnp.float32) + gbi_ref[...]
            gh = jnp.dot(mem, gwh_ref[...],
                         preferred_element_type=jnp.float32) + gbh_ref[...]
            r = _sigmoid(gi[:, :64] + gh[:, :64])
            z = _sigmoid(gi[:, 64:128] + gh[:, 64:128])
            n = jnp.tanh(gi[:, 128:] + r * gh[:, 128:])
            return (1.0 - z) * n + z * mem

        def mlp(xfull):
            h1 = jnp.maximum(
                jnp.dot(xfull.astype(jnp.bfloat16), mw1_ref[...],
                        preferred_element_type=jnp.float32) + mb1_ref[...],
                0.0)
            return (jnp.dot(h1.astype(jnp.bfloat16), mw2_ref[...],
                            preferred_element_type=jnp.float32) + mb2_ref[...])

        agg_s = agg_for(sc_ref[...])
        agg_d = agg_for(dc_ref[...])
        mem_s = gru(agg_s, sm_b[...])
        mem_d = gru(agg_d, dm_b[...])
        s_full = jnp.concatenate(
            [mem_s, ss_ref[...], se_ref[...], sy_ref[...]], axis=1)
        d_full = jnp.concatenate(
            [mem_d, ds_ref[...], de_ref[...], dy_ref[...]], axis=1)
        cat = jnp.concatenate([mlp(s_full), mlp(d_full), ea_b[...]], axis=1)
        out_ref[...] = (jnp.dot(cat, pw_ref[...],
                                preferred_element_type=jnp.float32)
                        + pb_ref[...])


def _fused(t2d, edge_attr, src_m, dst_m,
           src_col, dst_col, src_static, dst_static, src_dyn, dst_dyn,
           src_e, dst_e, src_row, dst_row, t_row,
           wt, bt, en_w1, en_b1, tflat, b0,
           gru_wi, gru_wh, gru_bi, gru_bh,
           mlp_w1, mlp_b1, mlp_w2, mlp_b2, pred_w, pred_b):
    grid = (N_A + N_B,)

    def tile_a(d):
        return pl.BlockSpec((TB_A, d),
                            lambda i: (jnp.where(i < N_A, i, 0), 0))

    def tile_b(d):
        return pl.BlockSpec((TB_B, d),
                            lambda i: (jnp.where(i < N_A, 0, i - N_A), 0))

    full = lambda r, c: pl.BlockSpec((r, c), lambda i: (0, 0))
    in_dim = MEMORY_DIM + STATIC_DIM + EMB_DIM + DYNAMIC_DIM
    return pl.pallas_call(
        _fused_body,
        grid=grid,
        in_specs=[
            tile_a(1), tile_a(EDGE_DIM), tile_a(MEMORY_DIM),
            tile_a(MEMORY_DIM),
            tile_b(1), tile_b(1), tile_b(MEMORY_DIM), tile_b(MEMORY_DIM),
            tile_b(STATIC_DIM), tile_b(STATIC_DIM),
            tile_b(DYNAMIC_DIM), tile_b(DYNAMIC_DIM),
            tile_b(EMB_DIM), tile_b(EMB_DIM), tile_b(EDGE_DIM),
            full(1, B), full(1, B), full(1, B),
            full(1, TIME_DIM), full(1, TIME_DIM),
            full(2 * TIME_DIM, 64), full(1, 64),
            full(64 * 128, MSG_DIM), full(2 * MEMORY_DIM, MSG_DIM),
            full(MSG_DIM, 3 * MEMORY_DIM), full(MEMORY_DIM, 3 * MEMORY_DIM),
            full(1, 3 * MEMORY_DIM), full(1, 3 * MEMORY_DIM),
            full(in_dim, NODE_DIM), full(1, NODE_DIM),
            full(NODE_DIM, NODE_DIM), full(1, NODE_DIM),
            full(2 * NODE_DIM + EDGE_DIM, 1), full(1, 1),
        ],
        out_specs=pl.BlockSpec((TB_B, 1),
                               lambda i: (jnp.where(i < N_A, 0, i - N_A), 0)),
        out_shape=jax.ShapeDtypeStruct((B, 1), jnp.float32),
        scratch_shapes=[pltpu.VMEM((2, B, MSG_DIM), jnp.bfloat16)],
    )(t2d, edge_attr, src_m, dst_m,
      src_col, dst_col, src_m, dst_m, src_static, dst_static,
      src_dyn, dst_dyn, src_e, dst_e, edge_attr,
      src_row, dst_row, t_row,
      wt, bt, en_w1, en_b1, tflat, b0,
      gru_wi, gru_wh, gru_bi, gru_bh,
      mlp_w1, mlp_b1, mlp_w2, mlp_b2, pred_w, pred_b)


def kernel(src, dst, t, edge_attr, src_static, dst_static, src_dynamic,
           dst_dynamic, memory, last_update, w_time, b_time, en_w1, en_b1,
           en_w2, en_b2, gru_wi, gru_wh, gru_bi, gru_bh, emb_table,
           mlp_w1, mlp_b1, mlp_w2, mlp_b2, pred_w, pred_b):
    src = src.astype(jnp.int32)
    dst = dst.astype(jnp.int32)
    t = t.astype(jnp.int32)

    tflat = en_w2.astype(jnp.bfloat16).reshape(64 * 128, MSG_DIM)
    b0 = en_b2.reshape(2 * MEMORY_DIM, MSG_DIM)

    src_m, dst_m, src_e, dst_e = _sc_gather(memory, emb_table, src, dst)

    pred = _fused(t.astype(jnp.float32).reshape(B, 1), edge_attr,
                  src_m, dst_m,
                  src.reshape(B, 1), dst.reshape(B, 1),
                  src_static, dst_static, src_dynamic, dst_dynamic,
                  src_e, dst_e,
                  src.reshape(1, B), dst.reshape(1, B), t.reshape(1, B),
                  w_time, b_time.reshape(1, TIME_DIM),
                  en_w1, en_b1.reshape(1, 64), tflat, b0,
                  gru_wi, gru_wh, gru_bi.reshape(1, 3 * MEMORY_DIM),
                  gru_bh.reshape(1, 3 * MEMORY_DIM),
                  mlp_w1.astype(jnp.bfloat16), mlp_b1.reshape(1, NODE_DIM),
                  mlp_w2.astype(jnp.bfloat16), mlp_b2.reshape(1, NODE_DIM),
                  pred_w, pred_b.reshape(1, 1))
    return pred
